# Initial kernel scaffold; baseline (speedup 1.0000x reference)
import jax
import jax.numpy as jnp
from jax.experimental import pallas as pl


def kernel(h_user, h_item, edge_clicks, edge_clickedby, W1c, b1c, W1cb, b1cb, W2c, b2c, W2cb, b2cb):
    raise NotImplementedError("write your pallas kernel here")


# speedup vs baseline: 4.0456x; 4.0456x over previous
import functools

import jax
import jax.numpy as jnp
from jax import lax
from jax.experimental import pallas as pl
from jax.experimental.pallas import tpu as pltpu
from jax.experimental.pallas import tpu_sc as plsc

N = 10000
E = 320000
D = 128
NP = 10240
NT = 16
K = 80
EPT = E // NT
RPT = 640
HPT = NP // NT

_mesh = plsc.VectorSubcoreMesh(core_axis_name="c", subcore_axis_name="s")


@functools.partial(
    pl.kernel,
    out_type=jax.ShapeDtypeStruct((4 * NP,), jnp.float32),
    mesh=_mesh,
    scratch_types=[
        pltpu.VMEM((K,), jnp.int32),
        pltpu.VMEM((K,), jnp.int32),
        pltpu.VMEM((K,), jnp.float32),
        pltpu.VMEM((HPT,), jnp.float32),
        pltpu.VMEM_SHARED((NP,), jnp.float32),
        pltpu.VMEM_SHARED((NP,), jnp.float32),
    ],
)
def _sc_degrees(src_hbm, dst_hbm, out_hbm, sidx, didx, ones, zbuf, hist_s, hist_d):
    ci = lax.axis_index("c")
    si = lax.axis_index("s")

    def fill(i, _):
        zbuf[pl.ds(i * 16, 16)] = jnp.zeros((16,), jnp.float32)
        return 0
    lax.fori_loop(0, HPT // 16, fill, 0)

    def fill1(i, _):
        ones[pl.ds(i * 16, 16)] = jnp.full((16,), 1.0, jnp.float32)
        return 0
    lax.fori_loop(0, K // 16, fill1, 0)

    pltpu.sync_copy(zbuf, hist_s.at[pl.ds(si * HPT, HPT)])
    pltpu.sync_copy(zbuf, hist_d.at[pl.ds(si * HPT, HPT)])
    plsc.subcore_barrier()

    def step(g, _):
        base = ci * E + si * EPT + g * K
        pltpu.sync_copy(src_hbm.at[pl.ds(base, K)], sidx)
        pltpu.sync_copy(dst_hbm.at[pl.ds(base, K)], didx)
        pltpu.sync_copy(ones, hist_s.at[sidx], add=True)
        pltpu.sync_copy(ones, hist_d.at[didx], add=True)
        return 0
    lax.fori_loop(0, EPT // K, step, 0)

    plsc.subcore_barrier()
    pltpu.sync_copy(hist_s.at[pl.ds(si * HPT, HPT)],
                    out_hbm.at[pl.ds((ci * 2 + 0) * NP + si * HPT, HPT)])
    pltpu.sync_copy(hist_d.at[pl.ds(si * HPT, HPT)],
                    out_hbm.at[pl.ds((ci * 2 + 1) * NP + si * HPT, HPT)])


@functools.partial(
    pl.kernel,
    out_type=jax.ShapeDtypeStruct((2 * N, D), jnp.float32),
    mesh=_mesh,
    scratch_types=[
        pltpu.VMEM((K,), jnp.int32),
        pltpu.VMEM((K,), jnp.int32),
        pltpu.VMEM((K, D), jnp.float32),
        pltpu.VMEM_SHARED((N, D), jnp.float32),
        pltpu.SemaphoreType.DMA,
    ],
)
def _sc_mp(hs_hbm, srcoff_hbm, dst_hbm, out_hbm, sidx, didx, rows, acc, sem):
    ci = lax.axis_index("c")
    si = lax.axis_index("s")

    def zrow(r, _):
        def zlane(j, _):
            rows[r, pl.ds(j * 16, 16)] = jnp.zeros((16,), jnp.float32)
            return 0
        return lax.fori_loop(0, D // 16, zlane, 0)
    lax.fori_loop(0, K, zrow, 0)

    row_lo = si * RPT
    nchunks = jnp.where(si == NT - 1, (N - (NT - 1) * RPT) // K, RPT // K)

    def zchunk(t, _):
        pltpu.sync_copy(rows, acc.at[pl.ds(row_lo + t * K, K)])
        return 0
    lax.fori_loop(0, nchunks, zchunk, 0)
    plsc.subcore_barrier()

    def step(g, _):
        base = ci * E + si * EPT + g * K
        pltpu.sync_copy(srcoff_hbm.at[pl.ds(base, K)], sidx)
        pltpu.sync_copy(dst_hbm.at[pl.ds(base, K)], didx)
        pltpu.async_copy(hs_hbm.at[sidx], rows, sem).wait()
        pltpu.sync_copy(rows, acc.at[didx], add=True)
        return 0
    lax.fori_loop(0, EPT // K, step, 0)

    plsc.subcore_barrier()

    def wchunk(t, _):
        pltpu.sync_copy(acc.at[pl.ds(row_lo + t * K, K)],
                        out_hbm.at[pl.ds(ci * N + row_lo + t * K, K)])
        return 0
    lax.fori_loop(0, nchunks, wchunk, 0)


_BR = 1000

def _rs(x):
    return lax.rsqrt(jnp.maximum(x, 1.0))


def _prescale_body(hu_ref, hi_ref, dcs_ref, dcbs_ref, hsu_ref, hsi_ref):
    hsu_ref[...] = hu_ref[...] * _rs(dcs_ref[...])
    hsi_ref[...] = hi_ref[...] * _rs(dcbs_ref[...])


def _prescale(hu, hi, dcs, dcbs):
    blk = lambda w: pl.BlockSpec((_BR, w), lambda i: (i, 0))
    return pl.pallas_call(
        _prescale_body,
        grid=(N // _BR,),
        in_specs=[blk(D), blk(D), blk(1), blk(1)],
        out_specs=[blk(D), blk(D)],
        out_shape=[jax.ShapeDtypeStruct((N, D), jnp.float32)] * 2,
    )(hu, hi, dcs, dcbs)


def _post1_body(aggi_ref, aggu_ref, dcd_ref, dcbd_ref, dcs_ref, dcbs_ref,
                wc_ref, wcb_ref, bc_ref, bcb_ref, hsu2_ref, hsi2_ref):
    ni = jax.nn.relu(
        jnp.dot(aggi_ref[...] * _rs(dcd_ref[...]), wc_ref[...],
                preferred_element_type=jnp.float32) + bc_ref[...])
    nu = jax.nn.relu(
        jnp.dot(aggu_ref[...] * _rs(dcbd_ref[...]), wcb_ref[...],
                preferred_element_type=jnp.float32) + bcb_ref[...])
    hsi2_ref[...] = ni * _rs(dcbs_ref[...])
    hsu2_ref[...] = nu * _rs(dcs_ref[...])


def _post1(aggi, aggu, dcd, dcbd, dcs, dcbs, wc, wcb, bc, bcb):
    blk = lambda w: pl.BlockSpec((_BR, w), lambda i: (i, 0))
    full = lambda s: pl.BlockSpec(s, lambda i: (0, 0))
    return pl.pallas_call(
        _post1_body,
        grid=(N // _BR,),
        in_specs=[blk(D), blk(D), blk(1), blk(1), blk(1), blk(1),
                  full((D, D)), full((D, D)), full((1, D)), full((1, D))],
        out_specs=[blk(D), blk(D)],
        out_shape=[jax.ShapeDtypeStruct((N, D), jnp.float32)] * 2,
    )(aggi, aggu, dcd, dcbd, dcs, dcbs, wc, wcb, bc, bcb)


def _post2_body(aggi_ref, aggu_ref, dcd_ref, dcbd_ref, wc_ref, wcb_ref,
                bc_ref, bcb_ref, hu0_ref, hi0_ref, ou_ref, oi_ref):
    ni = jax.nn.relu(
        jnp.dot(aggi_ref[...] * _rs(dcd_ref[...]), wc_ref[...],
                preferred_element_type=jnp.float32) + bc_ref[...])
    nu = jax.nn.relu(
        jnp.dot(aggu_ref[...] * _rs(dcbd_ref[...]), wcb_ref[...],
                preferred_element_type=jnp.float32) + bcb_ref[...])
    oi_ref[...] = jax.nn.relu(ni + hi0_ref[...])
    ou_ref[...] = jax.nn.relu(nu + hu0_ref[...])


def _post2(aggi, aggu, dcd, dcbd, wc, wcb, bc, bcb, hu0, hi0):
    blk = lambda w: pl.BlockSpec((_BR, w), lambda i: (i, 0))
    full = lambda s: pl.BlockSpec(s, lambda i: (0, 0))
    return pl.pallas_call(
        _post2_body,
        grid=(N // _BR,),
        in_specs=[blk(D), blk(D), blk(1), blk(1),
                  full((D, D)), full((D, D)), full((1, D)), full((1, D)),
                  blk(D), blk(D)],
        out_specs=[blk(D), blk(D)],
        out_shape=[jax.ShapeDtypeStruct((N, D), jnp.float32)] * 2,
    )(aggi, aggu, dcd, dcbd, wc, wcb, bc, bcb, hu0, hi0)


def kernel(h_user, h_item, edge_clicks, edge_clickedby,
           W1c, b1c, W1cb, b1cb, W2c, b2c, W2cb, b2cb):
    ec0, ec1 = edge_clicks[0], edge_clicks[1]
    ecb0, ecb1 = edge_clickedby[0], edge_clickedby[1]

    src_flat = jnp.concatenate([ec0, ecb0])
    dst_flat = jnp.concatenate([ec1, ecb1])
    deg = _sc_degrees(src_flat, dst_flat)
    dcs = deg[0 * NP:0 * NP + N].reshape(N, 1)
    dcd = deg[1 * NP:1 * NP + N].reshape(N, 1)
    dcbs = deg[2 * NP:2 * NP + N].reshape(N, 1)
    dcbd = deg[3 * NP:3 * NP + N].reshape(N, 1)

    b1c2, b1cb2 = b1c.reshape(1, D), b1cb.reshape(1, D)
    b2c2, b2cb2 = b2c.reshape(1, D), b2cb.reshape(1, D)

    srcoff = jnp.concatenate([ec0, ecb0 + N])
    dsts = dst_flat

    hs_u1, hs_i1 = _prescale(h_user, h_item, dcs, dcbs)
    agg1 = _sc_mp(jnp.concatenate([hs_u1, hs_i1], axis=0), srcoff, dsts)
    hs_u2, hs_i2 = _post1(agg1[:N], agg1[N:], dcd, dcbd, dcs, dcbs,
                          W1c, W1cb, b1c2, b1cb2)
    agg2 = _sc_mp(jnp.concatenate([hs_u2, hs_i2], axis=0), srcoff, dsts)
    out_user, out_item = _post2(agg2[:N], agg2[N:], dcd, dcbd,
                                W2c, W2cb, b2c2, b2cb2, h_user, h_item)
    return (out_user, out_item)


# speedup vs baseline: 8.0428x; 1.9880x over previous
import functools

import jax
import jax.numpy as jnp
from jax import lax
from jax.experimental import pallas as pl
from jax.experimental.pallas import tpu as pltpu
from jax.experimental.pallas import tpu_sc as plsc

N = 10000
E = 320000
D = 128
NP = 10240
NT = 16
K = 80
G = 250
EPT = E // NT
RPT = 640
HPT = NP // NT

_mesh = plsc.VectorSubcoreMesh(core_axis_name="c", subcore_axis_name="s")


@functools.partial(
    pl.kernel,
    out_type=jax.ShapeDtypeStruct((4 * NP,), jnp.float32),
    mesh=_mesh,
    scratch_types=[
        pltpu.VMEM((K,), jnp.int32),
        pltpu.VMEM((K,), jnp.int32),
        pltpu.VMEM((K,), jnp.int32),
        pltpu.VMEM((K,), jnp.int32),
        pltpu.VMEM((K,), jnp.float32),
        pltpu.VMEM((HPT,), jnp.float32),
        pltpu.VMEM_SHARED((NP,), jnp.float32),
        pltpu.VMEM_SHARED((NP,), jnp.float32),
        pltpu.SemaphoreType.DMA,
        pltpu.SemaphoreType.DMA,
    ],
)
def _sc_degrees(src_hbm, dst_hbm, out_hbm, sidx0, didx0, sidx1, didx1,
                ones, zbuf, hist_s, hist_d, semi0, semi1):
    ci = lax.axis_index("c")
    si = lax.axis_index("s")

    def fill(i, _):
        zbuf[pl.ds(i * 16, 16)] = jnp.zeros((16,), jnp.float32)
        return 0
    lax.fori_loop(0, HPT // 16, fill, 0)

    def fill1(i, _):
        ones[pl.ds(i * 16, 16)] = jnp.full((16,), 1.0, jnp.float32)
        return 0
    lax.fori_loop(0, K // 16, fill1, 0)

    pltpu.sync_copy(zbuf, hist_s.at[pl.ds(si * HPT, HPT)])
    pltpu.sync_copy(zbuf, hist_d.at[pl.ds(si * HPT, HPT)])
    plsc.subcore_barrier()

    tbase = ci * E + si * EPT

    def iload(g, sb, db, sem):
        pltpu.async_copy(src_hbm.at[pl.ds(tbase + g * K, K)], sb, sem)
        pltpu.async_copy(dst_hbm.at[pl.ds(tbase + g * K, K)], db, sem)

    def iwait(g, sb, db, sem):
        pltpu.make_async_copy(src_hbm.at[pl.ds(tbase + g * K, K)], sb, sem).wait()
        pltpu.make_async_copy(dst_hbm.at[pl.ds(tbase + g * K, K)], db, sem).wait()

    iload(0, sidx0, didx0, semi0)

    def step(gi, _):
        g0 = 2 * gi
        iwait(g0, sidx0, didx0, semi0)
        iload(g0 + 1, sidx1, didx1, semi1)
        pltpu.sync_copy(ones, hist_s.at[sidx0], add=True)
        pltpu.sync_copy(ones, hist_d.at[didx0], add=True)
        iwait(g0 + 1, sidx1, didx1, semi1)

        @pl.when(gi < G // 2 - 1)
        def _():
            iload(g0 + 2, sidx0, didx0, semi0)
        pltpu.sync_copy(ones, hist_s.at[sidx1], add=True)
        pltpu.sync_copy(ones, hist_d.at[didx1], add=True)
        return 0
    lax.fori_loop(0, G // 2, step, 0)

    plsc.subcore_barrier()
    pltpu.sync_copy(hist_s.at[pl.ds(si * HPT, HPT)],
                    out_hbm.at[pl.ds((ci * 2 + 0) * NP + si * HPT, HPT)])
    pltpu.sync_copy(hist_d.at[pl.ds(si * HPT, HPT)],
                    out_hbm.at[pl.ds((ci * 2 + 1) * NP + si * HPT, HPT)])


@functools.partial(
    pl.kernel,
    out_type=jax.ShapeDtypeStruct((2 * N, D), jnp.float32),
    mesh=_mesh,
    scratch_types=[
        pltpu.VMEM((K,), jnp.int32),
        pltpu.VMEM((K,), jnp.int32),
        pltpu.VMEM((K,), jnp.int32),
        pltpu.VMEM((K,), jnp.int32),
        pltpu.VMEM((K, D), jnp.float32),
        pltpu.VMEM((K, D), jnp.float32),
        pltpu.VMEM_SHARED((N, D), jnp.float32),
        pltpu.SemaphoreType.DMA,
        pltpu.SemaphoreType.DMA,
        pltpu.SemaphoreType.DMA,
        pltpu.SemaphoreType.DMA,
    ],
)
def _sc_mp(hs_hbm, srcoff_hbm, dst_hbm, out_hbm, sidx0, didx0, sidx1, didx1,
           rows0, rows1, acc, semi0, semi1, semg0, semg1):
    ci = lax.axis_index("c")
    si = lax.axis_index("s")

    def zrow(r, _):
        def zlane(j, _):
            rows0[r, pl.ds(j * 16, 16)] = jnp.zeros((16,), jnp.float32)
            return 0
        return lax.fori_loop(0, D // 16, zlane, 0)
    lax.fori_loop(0, K, zrow, 0)

    row_lo = si * RPT
    nchunks = jnp.where(si == NT - 1, (N - (NT - 1) * RPT) // K, RPT // K)

    def zchunk(t, _):
        pltpu.sync_copy(rows0, acc.at[pl.ds(row_lo + t * K, K)])
        return 0
    lax.fori_loop(0, nchunks, zchunk, 0)

    plsc.subcore_barrier()

    tbase = ci * E + si * EPT

    def iload(g, sb, db, sem):
        pltpu.async_copy(srcoff_hbm.at[pl.ds(tbase + g * K, K)], sb, sem)
        pltpu.async_copy(dst_hbm.at[pl.ds(tbase + g * K, K)], db, sem)

    def iwait(g, sb, db, sem):
        pltpu.make_async_copy(
            srcoff_hbm.at[pl.ds(tbase + g * K, K)], sb, sem).wait()
        pltpu.make_async_copy(
            dst_hbm.at[pl.ds(tbase + g * K, K)], db, sem).wait()

    iload(0, sidx0, didx0, semi0)
    iwait(0, sidx0, didx0, semi0)
    pltpu.async_copy(hs_hbm.at[sidx0], rows0, semg0)
    iload(1, sidx1, didx1, semi1)

    def step(gi, _):
        g0 = 2 * gi
        iwait(g0 + 1, sidx1, didx1, semi1)
        pltpu.make_async_copy(hs_hbm.at[sidx0], rows0, semg0).wait()
        pltpu.async_copy(hs_hbm.at[sidx1], rows1, semg1)
        pltpu.sync_copy(rows0, acc.at[didx0], add=True)

        @pl.when(gi < G // 2 - 1)
        def _():
            iload(g0 + 2, sidx0, didx0, semi0)
            iwait(g0 + 2, sidx0, didx0, semi0)
            pltpu.async_copy(hs_hbm.at[sidx0], rows0, semg0)
        pltpu.make_async_copy(hs_hbm.at[sidx1], rows1, semg1).wait()
        pltpu.sync_copy(rows1, acc.at[didx1], add=True)

        @pl.when(gi < G // 2 - 1)
        def _():
            iload(g0 + 3, sidx1, didx1, semi1)
        return 0
    lax.fori_loop(0, G // 2, step, 0)

    plsc.subcore_barrier()

    def wchunk(t, _):
        pltpu.sync_copy(acc.at[pl.ds(row_lo + t * K, K)],
                        out_hbm.at[pl.ds(ci * N + row_lo + t * K, K)])
        return 0
    lax.fori_loop(0, nchunks, wchunk, 0)


_BR = 1000

def _rs(x):
    return lax.rsqrt(jnp.maximum(x, 1.0))


def _prescale_body(hu_ref, hi_ref, dcs_ref, dcbs_ref, hsu_ref, hsi_ref):
    hsu_ref[...] = hu_ref[...] * _rs(dcs_ref[...])
    hsi_ref[...] = hi_ref[...] * _rs(dcbs_ref[...])


def _prescale(hu, hi, dcs, dcbs):
    blk = lambda w: pl.BlockSpec((_BR, w), lambda i: (i, 0))
    return pl.pallas_call(
        _prescale_body,
        grid=(N // _BR,),
        in_specs=[blk(D), blk(D), blk(1), blk(1)],
        out_specs=[blk(D), blk(D)],
        out_shape=[jax.ShapeDtypeStruct((N, D), jnp.float32)] * 2,
    )(hu, hi, dcs, dcbs)


def _post1_body(aggi_ref, aggu_ref, dcd_ref, dcbd_ref, dcs_ref, dcbs_ref,
                wc_ref, wcb_ref, bc_ref, bcb_ref, hsu2_ref, hsi2_ref):
    ni = jax.nn.relu(
        jnp.dot(aggi_ref[...] * _rs(dcd_ref[...]), wc_ref[...],
                preferred_element_type=jnp.float32) + bc_ref[...])
    nu = jax.nn.relu(
        jnp.dot(aggu_ref[...] * _rs(dcbd_ref[...]), wcb_ref[...],
                preferred_element_type=jnp.float32) + bcb_ref[...])
    hsi2_ref[...] = ni * _rs(dcbs_ref[...])
    hsu2_ref[...] = nu * _rs(dcs_ref[...])


def _post1(aggi, aggu, dcd, dcbd, dcs, dcbs, wc, wcb, bc, bcb):
    blk = lambda w: pl.BlockSpec((_BR, w), lambda i: (i, 0))
    full = lambda s: pl.BlockSpec(s, lambda i: (0, 0))
    return pl.pallas_call(
        _post1_body,
        grid=(N // _BR,),
        in_specs=[blk(D), blk(D), blk(1), blk(1), blk(1), blk(1),
                  full((D, D)), full((D, D)), full((1, D)), full((1, D))],
        out_specs=[blk(D), blk(D)],
        out_shape=[jax.ShapeDtypeStruct((N, D), jnp.float32)] * 2,
    )(aggi, aggu, dcd, dcbd, dcs, dcbs, wc, wcb, bc, bcb)


def _post2_body(aggi_ref, aggu_ref, dcd_ref, dcbd_ref, wc_ref, wcb_ref,
                bc_ref, bcb_ref, hu0_ref, hi0_ref, ou_ref, oi_ref):
    ni = jax.nn.relu(
        jnp.dot(aggi_ref[...] * _rs(dcd_ref[...]), wc_ref[...],
                preferred_element_type=jnp.float32) + bc_ref[...])
    nu = jax.nn.relu(
        jnp.dot(aggu_ref[...] * _rs(dcbd_ref[...]), wcb_ref[...],
                preferred_element_type=jnp.float32) + bcb_ref[...])
    oi_ref[...] = jax.nn.relu(ni + hi0_ref[...])
    ou_ref[...] = jax.nn.relu(nu + hu0_ref[...])


def _post2(aggi, aggu, dcd, dcbd, wc, wcb, bc, bcb, hu0, hi0):
    blk = lambda w: pl.BlockSpec((_BR, w), lambda i: (i, 0))
    full = lambda s: pl.BlockSpec(s, lambda i: (0, 0))
    return pl.pallas_call(
        _post2_body,
        grid=(N // _BR,),
        in_specs=[blk(D), blk(D), blk(1), blk(1),
                  full((D, D)), full((D, D)), full((1, D)), full((1, D)),
                  blk(D), blk(D)],
        out_specs=[blk(D), blk(D)],
        out_shape=[jax.ShapeDtypeStruct((N, D), jnp.float32)] * 2,
    )(aggi, aggu, dcd, dcbd, wc, wcb, bc, bcb, hu0, hi0)


def kernel(h_user, h_item, edge_clicks, edge_clickedby,
           W1c, b1c, W1cb, b1cb, W2c, b2c, W2cb, b2cb):
    ec0, ec1 = edge_clicks[0], edge_clicks[1]
    ecb0, ecb1 = edge_clickedby[0], edge_clickedby[1]

    src_flat = jnp.concatenate([ec0, ecb0])
    dst_flat = jnp.concatenate([ec1, ecb1])
    deg = _sc_degrees(src_flat, dst_flat)
    dcs = deg[0 * NP:0 * NP + N].reshape(N, 1)
    dcd = deg[1 * NP:1 * NP + N].reshape(N, 1)
    dcbs = deg[2 * NP:2 * NP + N].reshape(N, 1)
    dcbd = deg[3 * NP:3 * NP + N].reshape(N, 1)

    b1c2, b1cb2 = b1c.reshape(1, D), b1cb.reshape(1, D)
    b2c2, b2cb2 = b2c.reshape(1, D), b2cb.reshape(1, D)

    srcoff = jnp.concatenate([ec0, ecb0 + N])
    dsts = dst_flat

    hs_u1, hs_i1 = _prescale(h_user, h_item, dcs, dcbs)
    agg1 = _sc_mp(jnp.concatenate([hs_u1, hs_i1], axis=0), srcoff, dsts)
    hs_u2, hs_i2 = _post1(agg1[:N], agg1[N:], dcd, dcbd, dcs, dcbs,
                          W1c, W1cb, b1c2, b1cb2)
    agg2 = _sc_mp(jnp.concatenate([hs_u2, hs_i2], axis=0), srcoff, dsts)
    out_user, out_item = _post2(agg2[:N], agg2[N:], dcd, dcbd,
                                W2c, W2cb, b2c2, b2cb2, h_user, h_item)
    return (out_user, out_item)


# speedup vs baseline: 9.5763x; 1.1907x over previous
import functools

import jax
import jax.numpy as jnp
from jax import lax
from jax.experimental import pallas as pl
from jax.experimental.pallas import tpu as pltpu
from jax.experimental.pallas import tpu_sc as plsc

N = 10000
E = 320000
D = 128
NP = 10240
NT = 16
K = 80
KB = 128
NF = 156
TK = 32
NPAIR = NF // 2
EPT = E // NT
RPT = 640
HPT = NP // NT

_mesh = plsc.VectorSubcoreMesh(core_axis_name="c", subcore_axis_name="s")


@functools.partial(
    pl.kernel,
    out_type=jax.ShapeDtypeStruct((4 * NP,), jnp.float32),
    mesh=_mesh,
    scratch_types=[
        pltpu.VMEM((KB,), jnp.int32),
        pltpu.VMEM((KB,), jnp.int32),
        pltpu.VMEM((KB,), jnp.int32),
        pltpu.VMEM((KB,), jnp.int32),
        pltpu.VMEM((TK,), jnp.int32),
        pltpu.VMEM((TK,), jnp.int32),
        pltpu.VMEM((KB,), jnp.float32),
        pltpu.VMEM((HPT,), jnp.float32),
        pltpu.VMEM_SHARED((NP,), jnp.float32),
        pltpu.VMEM_SHARED((NP,), jnp.float32),
        pltpu.SemaphoreType.DMA,
        pltpu.SemaphoreType.DMA,
    ],
)
def _sc_degrees(src_hbm, dst_hbm, out_hbm, sidx0, didx0, sidx1, didx1,
                sidxt, didxt, ones, zbuf, hist_s, hist_d, semi0, semi1):
    ci = lax.axis_index("c")
    si = lax.axis_index("s")

    def fill(i, _):
        zbuf[pl.ds(i * 16, 16)] = jnp.zeros((16,), jnp.float32)
        return 0
    lax.fori_loop(0, HPT // 16, fill, 0)

    def fill1(i, _):
        ones[pl.ds(i * 16, 16)] = jnp.full((16,), 1.0, jnp.float32)
        return 0
    lax.fori_loop(0, KB // 16, fill1, 0)

    pltpu.sync_copy(zbuf, hist_s.at[pl.ds(si * HPT, HPT)])
    pltpu.sync_copy(zbuf, hist_d.at[pl.ds(si * HPT, HPT)])
    plsc.subcore_barrier()

    tbase = ci * E + si * EPT

    def iload(g, sb, db, sem):
        pltpu.async_copy(src_hbm.at[pl.ds(tbase + g * KB, KB)], sb, sem)
        pltpu.async_copy(dst_hbm.at[pl.ds(tbase + g * KB, KB)], db, sem)

    def iwait(g, sb, db, sem):
        pltpu.make_async_copy(src_hbm.at[pl.ds(tbase + g * KB, KB)], sb, sem).wait()
        pltpu.make_async_copy(dst_hbm.at[pl.ds(tbase + g * KB, KB)], db, sem).wait()

    iload(0, sidx0, didx0, semi0)

    def step(gi, _):
        g0 = 2 * gi
        iwait(g0, sidx0, didx0, semi0)
        iload(g0 + 1, sidx1, didx1, semi1)
        pltpu.sync_copy(ones, hist_s.at[sidx0], add=True)
        pltpu.sync_copy(ones, hist_d.at[didx0], add=True)
        iwait(g0 + 1, sidx1, didx1, semi1)

        @pl.when(gi < NPAIR - 1)
        def _():
            iload(g0 + 2, sidx0, didx0, semi0)
        pltpu.sync_copy(ones, hist_s.at[sidx1], add=True)
        pltpu.sync_copy(ones, hist_d.at[didx1], add=True)
        return 0
    lax.fori_loop(0, NPAIR, step, 0)

    tb = tbase + NF * KB
    pltpu.sync_copy(src_hbm.at[pl.ds(tb, TK)], sidxt)
    pltpu.sync_copy(dst_hbm.at[pl.ds(tb, TK)], didxt)
    pltpu.sync_copy(ones.at[pl.ds(0, TK)], hist_s.at[sidxt], add=True)
    pltpu.sync_copy(ones.at[pl.ds(0, TK)], hist_d.at[didxt], add=True)

    plsc.subcore_barrier()
    pltpu.sync_copy(hist_s.at[pl.ds(si * HPT, HPT)],
                    out_hbm.at[pl.ds((ci * 2 + 0) * NP + si * HPT, HPT)])
    pltpu.sync_copy(hist_d.at[pl.ds(si * HPT, HPT)],
                    out_hbm.at[pl.ds((ci * 2 + 1) * NP + si * HPT, HPT)])


@functools.partial(
    pl.kernel,
    out_type=jax.ShapeDtypeStruct((2 * N, D), jnp.float32),
    mesh=_mesh,
    scratch_types=[
        pltpu.VMEM((KB,), jnp.int32),
        pltpu.VMEM((KB,), jnp.int32),
        pltpu.VMEM((KB,), jnp.int32),
        pltpu.VMEM((KB,), jnp.int32),
        pltpu.VMEM((TK,), jnp.int32),
        pltpu.VMEM((TK,), jnp.int32),
        pltpu.VMEM((KB, D), jnp.float32),
        pltpu.VMEM((KB, D), jnp.float32),
        pltpu.VMEM_SHARED((N, D), jnp.float32),
        pltpu.SemaphoreType.DMA,
        pltpu.SemaphoreType.DMA,
        pltpu.SemaphoreType.DMA,
        pltpu.SemaphoreType.DMA,
    ],
)
def _sc_mp(hs_hbm, srcoff_hbm, dst_hbm, out_hbm, sidx0, didx0, sidx1, didx1,
           sidxt, didxt, rows0, rows1, acc, semi0, semi1, semg0, semg1):
    ci = lax.axis_index("c")
    si = lax.axis_index("s")

    def zrow(r, _):
        def zlane(j, _):
            rows0[r, pl.ds(j * 16, 16)] = jnp.zeros((16,), jnp.float32)
            return 0
        return lax.fori_loop(0, D // 16, zlane, 0)
    lax.fori_loop(0, K, zrow, 0)

    row_lo = si * RPT
    nchunks = jnp.where(si == NT - 1, (N - (NT - 1) * RPT) // K, RPT // K)

    def zchunk(t, _):
        pltpu.sync_copy(rows0.at[pl.ds(0, K)], acc.at[pl.ds(row_lo + t * K, K)])
        return 0
    lax.fori_loop(0, nchunks, zchunk, 0)

    plsc.subcore_barrier()

    tbase = ci * E + si * EPT

    def iload(g, sb, db, sem):
        pltpu.async_copy(srcoff_hbm.at[pl.ds(tbase + g * KB, KB)], sb, sem)
        pltpu.async_copy(dst_hbm.at[pl.ds(tbase + g * KB, KB)], db, sem)

    def iwait(g, sb, db, sem):
        pltpu.make_async_copy(
            srcoff_hbm.at[pl.ds(tbase + g * KB, KB)], sb, sem).wait()
        pltpu.make_async_copy(
            dst_hbm.at[pl.ds(tbase + g * KB, KB)], db, sem).wait()

    iload(0, sidx0, didx0, semi0)
    iwait(0, sidx0, didx0, semi0)
    pltpu.async_copy(hs_hbm.at[sidx0], rows0, semg0)
    iload(1, sidx1, didx1, semi1)

    def step(gi, _):
        g0 = 2 * gi
        iwait(g0 + 1, sidx1, didx1, semi1)
        pltpu.make_async_copy(hs_hbm.at[sidx0], rows0, semg0).wait()
        pltpu.async_copy(hs_hbm.at[sidx1], rows1, semg1)
        pltpu.sync_copy(rows0, acc.at[didx0], add=True)

        @pl.when(gi < NPAIR - 1)
        def _():
            iload(g0 + 2, sidx0, didx0, semi0)
            iwait(g0 + 2, sidx0, didx0, semi0)
            pltpu.async_copy(hs_hbm.at[sidx0], rows0, semg0)
        pltpu.make_async_copy(hs_hbm.at[sidx1], rows1, semg1).wait()
        pltpu.sync_copy(rows1, acc.at[didx1], add=True)

        @pl.when(gi < NPAIR - 1)
        def _():
            iload(g0 + 3, sidx1, didx1, semi1)
        return 0
    lax.fori_loop(0, NPAIR, step, 0)

    tb = tbase + NF * KB
    pltpu.sync_copy(srcoff_hbm.at[pl.ds(tb, TK)], sidxt)
    pltpu.sync_copy(dst_hbm.at[pl.ds(tb, TK)], didxt)
    pltpu.async_copy(hs_hbm.at[sidxt], rows0.at[pl.ds(0, TK)], semg0).wait()
    pltpu.sync_copy(rows0.at[pl.ds(0, TK)], acc.at[didxt], add=True)

    plsc.subcore_barrier()

    def wchunk(t, _):
        pltpu.sync_copy(acc.at[pl.ds(row_lo + t * K, K)],
                        out_hbm.at[pl.ds(ci * N + row_lo + t * K, K)])
        return 0
    lax.fori_loop(0, nchunks, wchunk, 0)


_BR = 1000

def _rs(x):
    return lax.rsqrt(jnp.maximum(x, 1.0))


def _prescale_body(hu_ref, hi_ref, dcs_ref, dcbs_ref, hsu_ref, hsi_ref):
    hsu_ref[...] = hu_ref[...] * _rs(dcs_ref[...])
    hsi_ref[...] = hi_ref[...] * _rs(dcbs_ref[...])


def _prescale(hu, hi, dcs, dcbs):
    blk = lambda w: pl.BlockSpec((_BR, w), lambda i: (i, 0))
    return pl.pallas_call(
        _prescale_body,
        grid=(N // _BR,),
        in_specs=[blk(D), blk(D), blk(1), blk(1)],
        out_specs=[blk(D), blk(D)],
        out_shape=[jax.ShapeDtypeStruct((N, D), jnp.float32)] * 2,
    )(hu, hi, dcs, dcbs)


def _post1_body(aggi_ref, aggu_ref, dcd_ref, dcbd_ref, dcs_ref, dcbs_ref,
                wc_ref, wcb_ref, bc_ref, bcb_ref, hsu2_ref, hsi2_ref):
    ni = jax.nn.relu(
        jnp.dot(aggi_ref[...] * _rs(dcd_ref[...]), wc_ref[...],
                preferred_element_type=jnp.float32) + bc_ref[...])
    nu = jax.nn.relu(
        jnp.dot(aggu_ref[...] * _rs(dcbd_ref[...]), wcb_ref[...],
                preferred_element_type=jnp.float32) + bcb_ref[...])
    hsi2_ref[...] = ni * _rs(dcbs_ref[...])
    hsu2_ref[...] = nu * _rs(dcs_ref[...])


def _post1(aggi, aggu, dcd, dcbd, dcs, dcbs, wc, wcb, bc, bcb):
    blk = lambda w: pl.BlockSpec((_BR, w), lambda i: (i, 0))
    full = lambda s: pl.BlockSpec(s, lambda i: (0, 0))
    return pl.pallas_call(
        _post1_body,
        grid=(N // _BR,),
        in_specs=[blk(D), blk(D), blk(1), blk(1), blk(1), blk(1),
                  full((D, D)), full((D, D)), full((1, D)), full((1, D))],
        out_specs=[blk(D), blk(D)],
        out_shape=[jax.ShapeDtypeStruct((N, D), jnp.float32)] * 2,
    )(aggi, aggu, dcd, dcbd, dcs, dcbs, wc, wcb, bc, bcb)


def _post2_body(aggi_ref, aggu_ref, dcd_ref, dcbd_ref, wc_ref, wcb_ref,
                bc_ref, bcb_ref, hu0_ref, hi0_ref, ou_ref, oi_ref):
    ni = jax.nn.relu(
        jnp.dot(aggi_ref[...] * _rs(dcd_ref[...]), wc_ref[...],
                preferred_element_type=jnp.float32) + bc_ref[...])
    nu = jax.nn.relu(
        jnp.dot(aggu_ref[...] * _rs(dcbd_ref[...]), wcb_ref[...],
                preferred_element_type=jnp.float32) + bcb_ref[...])
    oi_ref[...] = jax.nn.relu(ni + hi0_ref[...])
    ou_ref[...] = jax.nn.relu(nu + hu0_ref[...])


def _post2(aggi, aggu, dcd, dcbd, wc, wcb, bc, bcb, hu0, hi0):
    blk = lambda w: pl.BlockSpec((_BR, w), lambda i: (i, 0))
    full = lambda s: pl.BlockSpec(s, lambda i: (0, 0))
    return pl.pallas_call(
        _post2_body,
        grid=(N // _BR,),
        in_specs=[blk(D), blk(D), blk(1), blk(1),
                  full((D, D)), full((D, D)), full((1, D)), full((1, D)),
                  blk(D), blk(D)],
        out_specs=[blk(D), blk(D)],
        out_shape=[jax.ShapeDtypeStruct((N, D), jnp.float32)] * 2,
    )(aggi, aggu, dcd, dcbd, wc, wcb, bc, bcb, hu0, hi0)


def kernel(h_user, h_item, edge_clicks, edge_clickedby,
           W1c, b1c, W1cb, b1cb, W2c, b2c, W2cb, b2cb):
    ec0, ec1 = edge_clicks[0], edge_clicks[1]
    ecb0, ecb1 = edge_clickedby[0], edge_clickedby[1]

    src_flat = jnp.concatenate([ec0, ecb0])
    dst_flat = jnp.concatenate([ec1, ecb1])
    deg = _sc_degrees(src_flat, dst_flat)
    dcs = deg[0 * NP:0 * NP + N].reshape(N, 1)
    dcd = deg[1 * NP:1 * NP + N].reshape(N, 1)
    dcbs = deg[2 * NP:2 * NP + N].reshape(N, 1)
    dcbd = deg[3 * NP:3 * NP + N].reshape(N, 1)

    b1c2, b1cb2 = b1c.reshape(1, D), b1cb.reshape(1, D)
    b2c2, b2cb2 = b2c.reshape(1, D), b2cb.reshape(1, D)

    srcoff = jnp.concatenate([ec0, ecb0 + N])
    dsts = dst_flat

    hs_u1, hs_i1 = _prescale(h_user, h_item, dcs, dcbs)
    agg1 = _sc_mp(jnp.concatenate([hs_u1, hs_i1], axis=0), srcoff, dsts)
    hs_u2, hs_i2 = _post1(agg1[:N], agg1[N:], dcd, dcbd, dcs, dcbs,
                          W1c, W1cb, b1c2, b1cb2)
    agg2 = _sc_mp(jnp.concatenate([hs_u2, hs_i2], axis=0), srcoff, dsts)
    out_user, out_item = _post2(agg2[:N], agg2[N:], dcd, dcbd,
                                W2c, W2cb, b2c2, b2cb2, h_user, h_item)
    return (out_user, out_item)


# speedup vs baseline: 10.7328x; 1.1208x over previous
import functools

import jax
import jax.numpy as jnp
from jax import lax
from jax.experimental import pallas as pl
from jax.experimental.pallas import tpu as pltpu
from jax.experimental.pallas import tpu_sc as plsc

N = 10000
E = 320000
D = 128
NP = 10240
NT = 16
K = 80
KB = 128
NF = 156
TK = 32
NPAIR = NF // 2
EPT = E // NT
RPT = 640
HPT = NP // NT

_mesh = plsc.VectorSubcoreMesh(core_axis_name="c", subcore_axis_name="s")


@functools.partial(
    pl.kernel,
    out_type=jax.ShapeDtypeStruct((4 * NP,), jnp.float32),
    mesh=_mesh,
    scratch_types=[
        pltpu.VMEM((KB,), jnp.int32),
        pltpu.VMEM((KB,), jnp.int32),
        pltpu.VMEM((KB,), jnp.int32),
        pltpu.VMEM((KB,), jnp.int32),
        pltpu.VMEM((TK,), jnp.int32),
        pltpu.VMEM((TK,), jnp.int32),
        pltpu.VMEM((KB,), jnp.float32),
        pltpu.VMEM((HPT,), jnp.float32),
        pltpu.VMEM_SHARED((NP,), jnp.float32),
        pltpu.VMEM_SHARED((NP,), jnp.float32),
        pltpu.SemaphoreType.DMA,
        pltpu.SemaphoreType.DMA,
    ],
)
def _sc_degrees(src_hbm, dst_hbm, out_hbm, sidx0, didx0, sidx1, didx1,
                sidxt, didxt, ones, zbuf, hist_s, hist_d, semi0, semi1):
    ci = lax.axis_index("c")
    si = lax.axis_index("s")

    def fill(i, _):
        zbuf[pl.ds(i * 16, 16)] = jnp.zeros((16,), jnp.float32)
        return 0
    lax.fori_loop(0, HPT // 16, fill, 0)

    def fill1(i, _):
        ones[pl.ds(i * 16, 16)] = jnp.full((16,), 1.0, jnp.float32)
        return 0
    lax.fori_loop(0, KB // 16, fill1, 0)

    pltpu.sync_copy(zbuf, hist_s.at[pl.ds(si * HPT, HPT)])
    pltpu.sync_copy(zbuf, hist_d.at[pl.ds(si * HPT, HPT)])
    plsc.subcore_barrier()

    tbase = ci * E + si * EPT

    def iload(g, sb, db, sem):
        pltpu.async_copy(src_hbm.at[pl.ds(tbase + g * KB, KB)], sb, sem)
        pltpu.async_copy(dst_hbm.at[pl.ds(tbase + g * KB, KB)], db, sem)

    def iwait(g, sb, db, sem):
        pltpu.make_async_copy(src_hbm.at[pl.ds(tbase + g * KB, KB)], sb, sem).wait()
        pltpu.make_async_copy(dst_hbm.at[pl.ds(tbase + g * KB, KB)], db, sem).wait()

    iload(0, sidx0, didx0, semi0)

    def step(gi, _):
        g0 = 2 * gi
        iwait(g0, sidx0, didx0, semi0)
        iload(g0 + 1, sidx1, didx1, semi1)
        pltpu.sync_copy(ones, hist_s.at[sidx0], add=True)
        pltpu.sync_copy(ones, hist_d.at[didx0], add=True)
        iwait(g0 + 1, sidx1, didx1, semi1)

        @pl.when(gi < NPAIR - 1)
        def _():
            iload(g0 + 2, sidx0, didx0, semi0)
        pltpu.sync_copy(ones, hist_s.at[sidx1], add=True)
        pltpu.sync_copy(ones, hist_d.at[didx1], add=True)
        return 0
    lax.fori_loop(0, NPAIR, step, 0)

    tb = tbase + NF * KB
    pltpu.sync_copy(src_hbm.at[pl.ds(tb, TK)], sidxt)
    pltpu.sync_copy(dst_hbm.at[pl.ds(tb, TK)], didxt)
    pltpu.sync_copy(ones.at[pl.ds(0, TK)], hist_s.at[sidxt], add=True)
    pltpu.sync_copy(ones.at[pl.ds(0, TK)], hist_d.at[didxt], add=True)

    plsc.subcore_barrier()
    pltpu.sync_copy(hist_s.at[pl.ds(si * HPT, HPT)],
                    out_hbm.at[pl.ds((ci * 2 + 0) * NP + si * HPT, HPT)])
    pltpu.sync_copy(hist_d.at[pl.ds(si * HPT, HPT)],
                    out_hbm.at[pl.ds((ci * 2 + 1) * NP + si * HPT, HPT)])


@functools.partial(
    pl.kernel,
    out_type=jax.ShapeDtypeStruct((2 * N, D), jnp.float32),
    mesh=_mesh,
    scratch_types=[
        [pltpu.VMEM((KB,), jnp.int32)] * 4,
        [pltpu.VMEM((KB,), jnp.int32)] * 4,
        pltpu.VMEM((TK,), jnp.int32),
        pltpu.VMEM((TK,), jnp.int32),
        [pltpu.VMEM((KB, D), jnp.float32)] * 2,
        pltpu.VMEM_SHARED((N, D), jnp.float32),
        [pltpu.SemaphoreType.DMA] * 4,
        [pltpu.SemaphoreType.DMA] * 2,
        [pltpu.SemaphoreType.DMA] * 2,
    ],
)
def _sc_mp(hs_hbm, srcoff_hbm, dst_hbm, out_hbm, sidx, didx,
           sidxt, didxt, rows, acc, semi, semg, sems):
    ci = lax.axis_index("c")
    si = lax.axis_index("s")

    def zrow(r, _):
        def zlane(j, _):
            rows[0][r, pl.ds(j * 16, 16)] = jnp.zeros((16,), jnp.float32)
            return 0
        return lax.fori_loop(0, D // 16, zlane, 0)
    lax.fori_loop(0, K, zrow, 0)

    row_lo = si * RPT
    nchunks = jnp.where(si == NT - 1, (N - (NT - 1) * RPT) // K, RPT // K)

    def zchunk(t, _):
        pltpu.sync_copy(rows[0].at[pl.ds(0, K)],
                        acc.at[pl.ds(row_lo + t * K, K)])
        return 0
    lax.fori_loop(0, nchunks, zchunk, 0)

    plsc.subcore_barrier()

    tbase = ci * E + si * EPT

    def iload(g, p):
        pltpu.async_copy(srcoff_hbm.at[pl.ds(tbase + g * KB, KB)],
                         sidx[p], semi[p])
        pltpu.async_copy(dst_hbm.at[pl.ds(tbase + g * KB, KB)],
                         didx[p], semi[p])

    def iwait(g, p):
        pltpu.make_async_copy(srcoff_hbm.at[pl.ds(tbase + g * KB, KB)],
                              sidx[p], semi[p]).wait()
        pltpu.make_async_copy(dst_hbm.at[pl.ds(tbase + g * KB, KB)],
                              didx[p], semi[p]).wait()

    iload(0, 0)
    iload(1, 1)
    iwait(0, 0)
    pltpu.async_copy(hs_hbm.at[sidx[0]], rows[0], semg[0])
    iload(2, 2)

    def phase(g, p):
        q = p % 2
        qn = 1 - q

        @pl.when(g > 0)
        def _():
            pltpu.make_async_copy(
                rows[qn], acc.at[didx[(p + 3) % 4]], sems[qn]).wait()

        @pl.when(g < NF - 1)
        def _():
            iwait(g + 1, (p + 1) % 4)
            pltpu.async_copy(hs_hbm.at[sidx[(p + 1) % 4]], rows[qn],
                             semg[qn])
        pltpu.make_async_copy(hs_hbm.at[sidx[p]], rows[q], semg[q]).wait()
        pltpu.async_copy(rows[q], acc.at[didx[p]], sems[q], add=True)

        @pl.when(g < NF - 3)
        def _():
            iload(g + 3, (p + 3) % 4)

    def step(bi, _):
        for j in range(4):
            phase(4 * bi + j, j)
        return 0
    lax.fori_loop(0, NF // 4, step, 0)

    pltpu.make_async_copy(rows[1], acc.at[didx[3]], sems[1]).wait()
    tb = tbase + NF * KB
    pltpu.sync_copy(srcoff_hbm.at[pl.ds(tb, TK)], sidxt)
    pltpu.sync_copy(dst_hbm.at[pl.ds(tb, TK)], didxt)
    pltpu.async_copy(hs_hbm.at[sidxt], rows[0].at[pl.ds(0, TK)], semg[0]).wait()
    pltpu.sync_copy(rows[0].at[pl.ds(0, TK)], acc.at[didxt], add=True)

    plsc.subcore_barrier()

    def wchunk(t, _):
        pltpu.sync_copy(acc.at[pl.ds(row_lo + t * K, K)],
                        out_hbm.at[pl.ds(ci * N + row_lo + t * K, K)])
        return 0
    lax.fori_loop(0, nchunks, wchunk, 0)


_BR = 1000

def _rs(x):
    return lax.rsqrt(jnp.maximum(x, 1.0))


def _prescale_body(hu_ref, hi_ref, dcs_ref, dcbs_ref, hsu_ref, hsi_ref):
    hsu_ref[...] = hu_ref[...] * _rs(dcs_ref[...])
    hsi_ref[...] = hi_ref[...] * _rs(dcbs_ref[...])


def _prescale(hu, hi, dcs, dcbs):
    blk = lambda w: pl.BlockSpec((_BR, w), lambda i: (i, 0))
    return pl.pallas_call(
        _prescale_body,
        grid=(N // _BR,),
        in_specs=[blk(D), blk(D), blk(1), blk(1)],
        out_specs=[blk(D), blk(D)],
        out_shape=[jax.ShapeDtypeStruct((N, D), jnp.float32)] * 2,
    )(hu, hi, dcs, dcbs)


def _post1_body(aggi_ref, aggu_ref, dcd_ref, dcbd_ref, dcs_ref, dcbs_ref,
                wc_ref, wcb_ref, bc_ref, bcb_ref, hsu2_ref, hsi2_ref):
    ni = jax.nn.relu(
        jnp.dot(aggi_ref[...] * _rs(dcd_ref[...]), wc_ref[...],
                preferred_element_type=jnp.float32) + bc_ref[...])
    nu = jax.nn.relu(
        jnp.dot(aggu_ref[...] * _rs(dcbd_ref[...]), wcb_ref[...],
                preferred_element_type=jnp.float32) + bcb_ref[...])
    hsi2_ref[...] = ni * _rs(dcbs_ref[...])
    hsu2_ref[...] = nu * _rs(dcs_ref[...])


def _post1(aggi, aggu, dcd, dcbd, dcs, dcbs, wc, wcb, bc, bcb):
    blk = lambda w: pl.BlockSpec((_BR, w), lambda i: (i, 0))
    full = lambda s: pl.BlockSpec(s, lambda i: (0, 0))
    return pl.pallas_call(
        _post1_body,
        grid=(N // _BR,),
        in_specs=[blk(D), blk(D), blk(1), blk(1), blk(1), blk(1),
                  full((D, D)), full((D, D)), full((1, D)), full((1, D))],
        out_specs=[blk(D), blk(D)],
        out_shape=[jax.ShapeDtypeStruct((N, D), jnp.float32)] * 2,
    )(aggi, aggu, dcd, dcbd, dcs, dcbs, wc, wcb, bc, bcb)


def _post2_body(aggi_ref, aggu_ref, dcd_ref, dcbd_ref, wc_ref, wcb_ref,
                bc_ref, bcb_ref, hu0_ref, hi0_ref, ou_ref, oi_ref):
    ni = jax.nn.relu(
        jnp.dot(aggi_ref[...] * _rs(dcd_ref[...]), wc_ref[...],
                preferred_element_type=jnp.float32) + bc_ref[...])
    nu = jax.nn.relu(
        jnp.dot(aggu_ref[...] * _rs(dcbd_ref[...]), wcb_ref[...],
                preferred_element_type=jnp.float32) + bcb_ref[...])
    oi_ref[...] = jax.nn.relu(ni + hi0_ref[...])
    ou_ref[...] = jax.nn.relu(nu + hu0_ref[...])


def _post2(aggi, aggu, dcd, dcbd, wc, wcb, bc, bcb, hu0, hi0):
    blk = lambda w: pl.BlockSpec((_BR, w), lambda i: (i, 0))
    full = lambda s: pl.BlockSpec(s, lambda i: (0, 0))
    return pl.pallas_call(
        _post2_body,
        grid=(N // _BR,),
        in_specs=[blk(D), blk(D), blk(1), blk(1),
                  full((D, D)), full((D, D)), full((1, D)), full((1, D)),
                  blk(D), blk(D)],
        out_specs=[blk(D), blk(D)],
        out_shape=[jax.ShapeDtypeStruct((N, D), jnp.float32)] * 2,
    )(aggi, aggu, dcd, dcbd, wc, wcb, bc, bcb, hu0, hi0)


def kernel(h_user, h_item, edge_clicks, edge_clickedby,
           W1c, b1c, W1cb, b1cb, W2c, b2c, W2cb, b2cb):
    ec0, ec1 = edge_clicks[0], edge_clicks[1]
    ecb0, ecb1 = edge_clickedby[0], edge_clickedby[1]

    src_flat = jnp.concatenate([ec0, ecb0])
    dst_flat = jnp.concatenate([ec1, ecb1])
    deg = _sc_degrees(src_flat, dst_flat)
    dcs = deg[0 * NP:0 * NP + N].reshape(N, 1)
    dcd = deg[1 * NP:1 * NP + N].reshape(N, 1)
    dcbs = deg[2 * NP:2 * NP + N].reshape(N, 1)
    dcbd = deg[3 * NP:3 * NP + N].reshape(N, 1)

    b1c2, b1cb2 = b1c.reshape(1, D), b1cb.reshape(1, D)
    b2c2, b2cb2 = b2c.reshape(1, D), b2cb.reshape(1, D)

    srcoff = jnp.concatenate([ec0, ecb0 + N])
    dsts = dst_flat

    hs_u1, hs_i1 = _prescale(h_user, h_item, dcs, dcbs)
    agg1 = _sc_mp(jnp.concatenate([hs_u1, hs_i1], axis=0), srcoff, dsts)
    hs_u2, hs_i2 = _post1(agg1[:N], agg1[N:], dcd, dcbd, dcs, dcbs,
                          W1c, W1cb, b1c2, b1cb2)
    agg2 = _sc_mp(jnp.concatenate([hs_u2, hs_i2], axis=0), srcoff, dsts)
    out_user, out_item = _post2(agg2[:N], agg2[N:], dcd, dcbd,
                                W2c, W2cb, b2c2, b2cb2, h_user, h_item)
    return (out_user, out_item)


# speedup vs baseline: 11.6745x; 1.0877x over previous
import functools

import jax
import jax.numpy as jnp
from jax import lax
from jax.experimental import pallas as pl
from jax.experimental.pallas import tpu as pltpu
from jax.experimental.pallas import tpu_sc as plsc

N = 10000
E = 320000
D = 128
NP = 10240
NT = 16
K = 80
KB = 128
NF = 156
TK = 32
NPAIR = NF // 2
EPT = E // NT
RPT = 640
HPT = NP // NT

_mesh = plsc.VectorSubcoreMesh(core_axis_name="c", subcore_axis_name="s")


@functools.partial(
    pl.kernel,
    out_type=jax.ShapeDtypeStruct((4 * NP,), jnp.float32),
    mesh=_mesh,
    scratch_types=[
        [pltpu.VMEM((KB,), jnp.int32)] * 4,
        [pltpu.VMEM((KB,), jnp.int32)] * 4,
        pltpu.VMEM((TK,), jnp.int32),
        pltpu.VMEM((TK,), jnp.int32),
        pltpu.VMEM((KB,), jnp.float32),
        pltpu.VMEM((HPT,), jnp.float32),
        pltpu.VMEM_SHARED((NP,), jnp.float32),
        pltpu.VMEM_SHARED((NP,), jnp.float32),
        [pltpu.SemaphoreType.DMA] * 4,
        [pltpu.SemaphoreType.DMA] * 2,
    ],
)
def _sc_degrees(src_hbm, dst_hbm, out_hbm, sidx, didx,
                sidxt, didxt, ones, zbuf, hist_s, hist_d, semi, sems):
    ci = lax.axis_index("c")
    si = lax.axis_index("s")

    def fill(i, _):
        zbuf[pl.ds(i * 16, 16)] = jnp.zeros((16,), jnp.float32)
        return 0
    lax.fori_loop(0, HPT // 16, fill, 0)

    def fill1(i, _):
        ones[pl.ds(i * 16, 16)] = jnp.full((16,), 1.0, jnp.float32)
        return 0
    lax.fori_loop(0, KB // 16, fill1, 0)

    pltpu.sync_copy(zbuf, hist_s.at[pl.ds(si * HPT, HPT)])
    pltpu.sync_copy(zbuf, hist_d.at[pl.ds(si * HPT, HPT)])
    plsc.subcore_barrier()

    tbase = ci * E + si * EPT

    def iload(g, p):
        pltpu.async_copy(src_hbm.at[pl.ds(tbase + g * KB, KB)],
                         sidx[p], semi[p])
        pltpu.async_copy(dst_hbm.at[pl.ds(tbase + g * KB, KB)],
                         didx[p], semi[p])

    def iwait(g, p):
        pltpu.make_async_copy(src_hbm.at[pl.ds(tbase + g * KB, KB)],
                              sidx[p], semi[p]).wait()
        pltpu.make_async_copy(dst_hbm.at[pl.ds(tbase + g * KB, KB)],
                              didx[p], semi[p]).wait()

    iload(0, 0)
    iload(1, 1)
    iload(2, 2)

    def phase(g, p):
        q = p % 2
        qn = 1 - q

        @pl.when(g > 0)
        def _():
            pltpu.make_async_copy(
                ones, hist_s.at[sidx[(p + 3) % 4]], sems[qn]).wait()
            pltpu.make_async_copy(
                ones, hist_d.at[didx[(p + 3) % 4]], sems[qn]).wait()
        iwait(g, p)
        pltpu.async_copy(ones, hist_s.at[sidx[p]], sems[q], add=True)
        pltpu.async_copy(ones, hist_d.at[didx[p]], sems[q], add=True)

        @pl.when(g < NF - 3)
        def _():
            iload(g + 3, (p + 3) % 4)

    def step(bi, _):
        for j in range(4):
            phase(4 * bi + j, j)
        return 0
    lax.fori_loop(0, NF // 4, step, 0)
    pltpu.make_async_copy(ones, hist_s.at[sidx[3]], sems[1]).wait()
    pltpu.make_async_copy(ones, hist_d.at[didx[3]], sems[1]).wait()

    tb = tbase + NF * KB
    pltpu.sync_copy(src_hbm.at[pl.ds(tb, TK)], sidxt)
    pltpu.sync_copy(dst_hbm.at[pl.ds(tb, TK)], didxt)
    pltpu.sync_copy(ones.at[pl.ds(0, TK)], hist_s.at[sidxt], add=True)
    pltpu.sync_copy(ones.at[pl.ds(0, TK)], hist_d.at[didxt], add=True)

    plsc.subcore_barrier()
    pltpu.sync_copy(hist_s.at[pl.ds(si * HPT, HPT)],
                    out_hbm.at[pl.ds((ci * 2 + 0) * NP + si * HPT, HPT)])
    pltpu.sync_copy(hist_d.at[pl.ds(si * HPT, HPT)],
                    out_hbm.at[pl.ds((ci * 2 + 1) * NP + si * HPT, HPT)])


@functools.partial(
    pl.kernel,
    out_type=jax.ShapeDtypeStruct((2 * N, D), jnp.float32),
    mesh=_mesh,
    scratch_types=[
        [pltpu.VMEM((KB,), jnp.int32)] * 4,
        [pltpu.VMEM((KB,), jnp.int32)] * 4,
        pltpu.VMEM((TK,), jnp.int32),
        pltpu.VMEM((TK,), jnp.int32),
        [pltpu.VMEM((KB, D), jnp.float32)] * 2,
        pltpu.VMEM_SHARED((N, D), jnp.float32),
        [pltpu.SemaphoreType.DMA] * 4,
        [pltpu.SemaphoreType.DMA] * 2,
        [pltpu.SemaphoreType.DMA] * 2,
    ],
)
def _sc_mp(hs_hbm, srcoff_hbm, dst_hbm, out_hbm, sidx, didx,
           sidxt, didxt, rows, acc, semi, semg, sems):
    ci = lax.axis_index("c")
    si = lax.axis_index("s")

    def zrow(r, _):
        def zlane(j, _):
            rows[0][r, pl.ds(j * 16, 16)] = jnp.zeros((16,), jnp.float32)
            return 0
        return lax.fori_loop(0, D // 16, zlane, 0)
    lax.fori_loop(0, K, zrow, 0)

    row_lo = si * RPT
    nchunks = jnp.where(si == NT - 1, (N - (NT - 1) * RPT) // K, RPT // K)

    def zchunk(t, _):
        pltpu.sync_copy(rows[0].at[pl.ds(0, K)],
                        acc.at[pl.ds(row_lo + t * K, K)])
        return 0
    lax.fori_loop(0, nchunks, zchunk, 0)

    plsc.subcore_barrier()

    tbase = ci * E + si * EPT

    def iload(g, p):
        pltpu.async_copy(srcoff_hbm.at[pl.ds(tbase + g * KB, KB)],
                         sidx[p], semi[p])
        pltpu.async_copy(dst_hbm.at[pl.ds(tbase + g * KB, KB)],
                         didx[p], semi[p])

    def iwait(g, p):
        pltpu.make_async_copy(srcoff_hbm.at[pl.ds(tbase + g * KB, KB)],
                              sidx[p], semi[p]).wait()
        pltpu.make_async_copy(dst_hbm.at[pl.ds(tbase + g * KB, KB)],
                              didx[p], semi[p]).wait()

    iload(0, 0)
    iload(1, 1)
    iwait(0, 0)
    pltpu.async_copy(hs_hbm.at[sidx[0]], rows[0], semg[0])
    iload(2, 2)

    def phase(g, p):
        q = p % 2
        qn = 1 - q

        @pl.when(g > 0)
        def _():
            pltpu.make_async_copy(
                rows[qn], acc.at[didx[(p + 3) % 4]], sems[qn]).wait()

        @pl.when(g < NF - 1)
        def _():
            iwait(g + 1, (p + 1) % 4)
            pltpu.async_copy(hs_hbm.at[sidx[(p + 1) % 4]], rows[qn],
                             semg[qn])
        pltpu.make_async_copy(hs_hbm.at[sidx[p]], rows[q], semg[q]).wait()
        pltpu.async_copy(rows[q], acc.at[didx[p]], sems[q], add=True)

        @pl.when(g < NF - 3)
        def _():
            iload(g + 3, (p + 3) % 4)

    def step(bi, _):
        for j in range(4):
            phase(4 * bi + j, j)
        return 0
    lax.fori_loop(0, NF // 4, step, 0)

    pltpu.make_async_copy(rows[1], acc.at[didx[3]], sems[1]).wait()
    tb = tbase + NF * KB
    pltpu.sync_copy(srcoff_hbm.at[pl.ds(tb, TK)], sidxt)
    pltpu.sync_copy(dst_hbm.at[pl.ds(tb, TK)], didxt)
    pltpu.async_copy(hs_hbm.at[sidxt], rows[0].at[pl.ds(0, TK)], semg[0]).wait()
    pltpu.sync_copy(rows[0].at[pl.ds(0, TK)], acc.at[didxt], add=True)

    plsc.subcore_barrier()

    def wchunk(t, _):
        pltpu.sync_copy(acc.at[pl.ds(row_lo + t * K, K)],
                        out_hbm.at[pl.ds(ci * N + row_lo + t * K, K)])
        return 0
    lax.fori_loop(0, nchunks, wchunk, 0)


_BR = 1000

def _rs(x):
    return lax.rsqrt(jnp.maximum(x, 1.0))


def _prescale_body(hu_ref, hi_ref, dcs_ref, dcbs_ref, hsu_ref, hsi_ref):
    hsu_ref[...] = hu_ref[...] * _rs(dcs_ref[...])
    hsi_ref[...] = hi_ref[...] * _rs(dcbs_ref[...])


def _prescale(hu, hi, dcs, dcbs):
    blk = lambda w: pl.BlockSpec((_BR, w), lambda i: (i, 0))
    return pl.pallas_call(
        _prescale_body,
        grid=(N // _BR,),
        in_specs=[blk(D), blk(D), blk(1), blk(1)],
        out_specs=[blk(D), blk(D)],
        out_shape=[jax.ShapeDtypeStruct((N, D), jnp.float32)] * 2,
    )(hu, hi, dcs, dcbs)


def _post1_body(aggi_ref, aggu_ref, dcd_ref, dcbd_ref, dcs_ref, dcbs_ref,
                wc_ref, wcb_ref, bc_ref, bcb_ref, hsu2_ref, hsi2_ref):
    ni = jax.nn.relu(
        jnp.dot(aggi_ref[...] * _rs(dcd_ref[...]), wc_ref[...],
                preferred_element_type=jnp.float32) + bc_ref[...])
    nu = jax.nn.relu(
        jnp.dot(aggu_ref[...] * _rs(dcbd_ref[...]), wcb_ref[...],
                preferred_element_type=jnp.float32) + bcb_ref[...])
    hsi2_ref[...] = ni * _rs(dcbs_ref[...])
    hsu2_ref[...] = nu * _rs(dcs_ref[...])


def _post1(aggi, aggu, dcd, dcbd, dcs, dcbs, wc, wcb, bc, bcb):
    blk = lambda w: pl.BlockSpec((_BR, w), lambda i: (i, 0))
    full = lambda s: pl.BlockSpec(s, lambda i: (0, 0))
    return pl.pallas_call(
        _post1_body,
        grid=(N // _BR,),
        in_specs=[blk(D), blk(D), blk(1), blk(1), blk(1), blk(1),
                  full((D, D)), full((D, D)), full((1, D)), full((1, D))],
        out_specs=[blk(D), blk(D)],
        out_shape=[jax.ShapeDtypeStruct((N, D), jnp.float32)] * 2,
    )(aggi, aggu, dcd, dcbd, dcs, dcbs, wc, wcb, bc, bcb)


def _post2_body(aggi_ref, aggu_ref, dcd_ref, dcbd_ref, wc_ref, wcb_ref,
                bc_ref, bcb_ref, hu0_ref, hi0_ref, ou_ref, oi_ref):
    ni = jax.nn.relu(
        jnp.dot(aggi_ref[...] * _rs(dcd_ref[...]), wc_ref[...],
                preferred_element_type=jnp.float32) + bc_ref[...])
    nu = jax.nn.relu(
        jnp.dot(aggu_ref[...] * _rs(dcbd_ref[...]), wcb_ref[...],
                preferred_element_type=jnp.float32) + bcb_ref[...])
    oi_ref[...] = jax.nn.relu(ni + hi0_ref[...])
    ou_ref[...] = jax.nn.relu(nu + hu0_ref[...])


def _post2(aggi, aggu, dcd, dcbd, wc, wcb, bc, bcb, hu0, hi0):
    blk = lambda w: pl.BlockSpec((_BR, w), lambda i: (i, 0))
    full = lambda s: pl.BlockSpec(s, lambda i: (0, 0))
    return pl.pallas_call(
        _post2_body,
        grid=(N // _BR,),
        in_specs=[blk(D), blk(D), blk(1), blk(1),
                  full((D, D)), full((D, D)), full((1, D)), full((1, D)),
                  blk(D), blk(D)],
        out_specs=[blk(D), blk(D)],
        out_shape=[jax.ShapeDtypeStruct((N, D), jnp.float32)] * 2,
    )(aggi, aggu, dcd, dcbd, wc, wcb, bc, bcb, hu0, hi0)


def kernel(h_user, h_item, edge_clicks, edge_clickedby,
           W1c, b1c, W1cb, b1cb, W2c, b2c, W2cb, b2cb):
    ec0, ec1 = edge_clicks[0], edge_clicks[1]
    ecb0, ecb1 = edge_clickedby[0], edge_clickedby[1]

    src_flat = jnp.concatenate([ec0, ecb0])
    dst_flat = jnp.concatenate([ec1, ecb1])
    deg = _sc_degrees(src_flat, dst_flat)
    dcs = deg[0 * NP:0 * NP + N].reshape(N, 1)
    dcd = deg[1 * NP:1 * NP + N].reshape(N, 1)
    dcbs = deg[2 * NP:2 * NP + N].reshape(N, 1)
    dcbd = deg[3 * NP:3 * NP + N].reshape(N, 1)

    b1c2, b1cb2 = b1c.reshape(1, D), b1cb.reshape(1, D)
    b2c2, b2cb2 = b2c.reshape(1, D), b2cb.reshape(1, D)

    srcoff = jnp.concatenate([ec0, ecb0 + N])
    dsts = dst_flat

    hs_u1, hs_i1 = _prescale(h_user, h_item, dcs, dcbs)
    agg1 = _sc_mp(jnp.concatenate([hs_u1, hs_i1], axis=0), srcoff, dsts)
    hs_u2, hs_i2 = _post1(agg1[:N], agg1[N:], dcd, dcbd, dcs, dcbs,
                          W1c, W1cb, b1c2, b1cb2)
    agg2 = _sc_mp(jnp.concatenate([hs_u2, hs_i2], axis=0), srcoff, dsts)
    out_user, out_item = _post2(agg2[:N], agg2[N:], dcd, dcbd,
                                W2c, W2cb, b2c2, b2cb2, h_user, h_item)
    return (out_user, out_item)
